# Initial kernel scaffold; baseline (speedup 1.0000x reference)
import jax
import jax.numpy as jnp
from jax.experimental import pallas as pl


def kernel(p, x, params):
    raise NotImplementedError("write your pallas kernel here")


# speedup vs baseline: 2.2521x; 2.2521x over previous
import functools
import math

import jax
import jax.numpy as jnp
from jax import lax
from jax.experimental import pallas as pl
from jax.experimental.pallas import tpu as pltpu

_PREC = lax.Precision.DEFAULT


def _dot(a, b):
    return lax.dot_general(a, b, (((a.ndim - 1,), (0,)), ((), ())),
                           precision=_PREC, preferred_element_type=jnp.float32)


def _pow2_floor(v):
    return 1 << (int(v).bit_length() - 1)


def _lin_kernel(x_ref, w_ref, b_ref, g_ref, bt_ref, o_ref, *, relu):
    y = _dot(x_ref[...], w_ref[...])
    y = (y + b_ref[...]) * g_ref[...] + bt_ref[...]
    if relu:
        y = jnp.maximum(y, 0.0)
    o_ref[...] = y


def _lin_res_kernel(x_ref, w_ref, b_ref, g_ref, bt_ref, r_ref, o_ref, *, relu):
    y = _dot(x_ref[...], w_ref[...])
    y = (y + b_ref[...]) * g_ref[...] + bt_ref[...] + r_ref[...]
    if relu:
        y = jnp.maximum(y, 0.0)
    o_ref[...] = y


def _lin_call(x, w, b=None, bn=None, relu=False, res=None):
    lead = x.shape[:-1]
    cin = x.shape[-1]
    cout = w.shape[-1]
    rows = int(math.prod(lead))
    x2 = x.reshape(rows, cin)
    rb = min(rows, 1024)
    b2 = jnp.zeros((1, cout), jnp.float32) if b is None else b.reshape(1, cout)
    if bn is None:
        g2 = jnp.ones((1, cout), jnp.float32)
        bt2 = jnp.zeros((1, cout), jnp.float32)
    else:
        g2 = bn['g'].reshape(1, cout)
        bt2 = bn['b'].reshape(1, cout)
    full = pl.BlockSpec((1, cout), lambda i: (0, 0))
    wspec = pl.BlockSpec((cin, cout), lambda i: (0, 0))
    rspec = pl.BlockSpec((rb, cout), lambda i: (i, 0))
    xspec = pl.BlockSpec((rb, cin), lambda i: (i, 0))
    if res is None:
        fn = functools.partial(_lin_kernel, relu=relu)
        args = (x2, w, b2, g2, bt2)
        specs = [xspec, wspec, full, full, full]
    else:
        fn = functools.partial(_lin_res_kernel, relu=relu)
        args = (x2, w, b2, g2, bt2, res.reshape(rows, cout))
        specs = [xspec, wspec, full, full, full, rspec]
    y = pl.pallas_call(
        fn,
        grid=(rows // rb,),
        in_specs=specs,
        out_specs=rspec,
        out_shape=jax.ShapeDtypeStruct((rows, cout), jnp.float32),
    )(*args)
    return y.reshape(lead + (cout,))


def _knn_kernel(q_ref, st_ref, idx_ref, dst_ref, *, k, ns):
    q = q_ref[0]
    st = st_ref[0]
    qq = jnp.sum(q * q, axis=1, keepdims=True)
    ss = jnp.sum(st * st, axis=0, keepdims=True)
    d = qq + ss - 2.0 * _dot(q, st)
    iota = lax.broadcasted_iota(jnp.int32, d.shape, 1)
    for j in range(k):
        m = jnp.min(d, axis=1, keepdims=True)
        ij = jnp.min(jnp.where(d == m, iota, ns), axis=1, keepdims=True)
        idx_ref[0, :, j:j + 1] = ij
        dst_ref[0, :, j:j + 1] = jnp.maximum(m, 0.0)
        if j + 1 < k:
            d = jnp.where(iota == ij, jnp.inf, d)


def _knn_call(q, s, k):
    bsz, nq, _ = q.shape
    ns = s.shape[1]
    st = jnp.swapaxes(s, 1, 2)
    qb = min(nq, max(64, _pow2_floor((1 << 20) // ns)))
    idx, dst = pl.pallas_call(
        functools.partial(_knn_kernel, k=k, ns=ns),
        grid=(bsz, nq // qb),
        in_specs=[
            pl.BlockSpec((1, qb, 3), lambda b, i: (b, i, 0)),
            pl.BlockSpec((1, 3, ns), lambda b, i: (b, 0, 0)),
        ],
        out_specs=[
            pl.BlockSpec((1, qb, k), lambda b, i: (b, i, 0)),
            pl.BlockSpec((1, qb, k), lambda b, i: (b, i, 0)),
        ],
        out_shape=[
            jax.ShapeDtypeStruct((bsz, nq, k), jnp.int32),
            jax.ShapeDtypeStruct((bsz, nq, k), jnp.float32),
        ],
    )(q, st)
    return idx, dst


def _fps_kernel(p_ref, o_ref, *, bsz, m, n8):
    xyz = [[p_ref[b, c] for c in range(3)] for b in range(bsz)]
    fiota = (lax.broadcasted_iota(jnp.int32, (8, n8), 0) * n8
             + lax.broadcasted_iota(jnp.int32, (8, n8), 1))
    n = 8 * n8

    def coord(v, nxt):
        return jnp.sum(jnp.where(fiota == nxt, v, 0.0))

    last0 = [[coord(xyz[b][c], 0) for c in range(3)] for b in range(bsz)]
    for b in range(bsz):
        o_ref[b, 0] = 0

    def body(i, st):
        last, dd = st
        new_last = []
        new_dd = []
        for b in range(bsz):
            dx = xyz[b][0] - last[b][0]
            dy = xyz[b][1] - last[b][1]
            dz = xyz[b][2] - last[b][2]
            d = (dx * dx + dy * dy) + dz * dz
            db = jnp.minimum(dd[b], d)
            mx = jnp.max(db)
            nxt = jnp.min(jnp.where(db == mx, fiota, n))
            o_ref[b, i] = nxt
            new_last.append([coord(xyz[b][c], nxt) for c in range(3)])
            new_dd.append(db)
        return new_last, new_dd

    dd0 = [jnp.full((8, n8), 1e10, jnp.float32) for _ in range(bsz)]
    lax.fori_loop(1, m, body, (last0, dd0))


def _fps_call(p, m):
    bsz, n, _ = p.shape
    n8 = n // 8
    pt = jnp.swapaxes(p, 1, 2).reshape(bsz, 3, 8, n8)
    return pl.pallas_call(
        functools.partial(_fps_kernel, bsz=bsz, m=m, n8=n8),
        in_specs=[pl.BlockSpec(memory_space=pltpu.VMEM)],
        out_specs=pl.BlockSpec(memory_space=pltpu.SMEM),
        out_shape=jax.ShapeDtypeStruct((bsz, m), jnp.int32),
    )(pt)


def _attn_kernel(xq_ref, xk_ref, xv_ref, rel_ref,
                 p1w_ref, p1b_ref, pg_ref, pb_ref, p2w_ref, p2b_ref,
                 g1_ref, t1_ref, w1w_ref, w1b_ref,
                 g2_ref, t2_ref, w2w_ref, w2b_ref,
                 bg_ref, bb_ref, o_ref, *, k, c, share):
    rb = o_ref.shape[0]
    rbk = rb * k
    c8 = c // share
    rel = rel_ref[...]
    t = _dot(rel, p1w_ref[...]) + p1b_ref[...]
    t = jnp.maximum(t * pg_ref[...] + pb_ref[...], 0.0)
    pe = _dot(t, p2w_ref[...]) + p2b_ref[...]
    xq = xq_ref[...]
    xk = xk_ref[...]
    w3 = (xk.reshape(rb, k, c) - xq.reshape(rb, 1, c)
          + pe.reshape(rb, k, c))
    w = w3.reshape(rbk, c)
    w = jnp.maximum(w * g1_ref[...] + t1_ref[...], 0.0)
    w = _dot(w, w1w_ref[...]) + w1b_ref[...]
    w = jnp.maximum(w * g2_ref[...] + t2_ref[...], 0.0)
    w = _dot(w, w2w_ref[...]) + w2b_ref[...]
    w3 = w.reshape(rb, k, c8)
    w3 = w3 - jnp.max(w3, axis=1, keepdims=True)
    e = jnp.exp(w3)
    sm = e / jnp.sum(e, axis=1, keepdims=True)
    v3 = (xv_ref[...] + pe).reshape(rb, k, c)
    wfull = jnp.concatenate([sm] * share, axis=2)
    o = jnp.sum(v3 * wfull, axis=1)
    o_ref[...] = jnp.maximum(o * bg_ref[...] + bb_ref[...], 0.0)


def _attn_call(xq, gk, gv, rel, pr, bn2, share, k):
    bsz, n, c = xq.shape
    rows = bsz * n
    c8 = c // share
    rb = min(rows, max(64, _pow2_floor((1 << 19) // (k * c))))
    xq2 = xq.reshape(rows, c)
    xk2 = gk.reshape(rows * k, c)
    xv2 = gv.reshape(rows * k, c)
    rel2 = rel.reshape(rows * k, 3)

    def row(v):
        return v.reshape(1, -1)

    full = lambda shape: pl.BlockSpec(shape, lambda i: (0, 0))
    args = (xq2, xk2, xv2, rel2,
            pr['p1']['W'], row(pr['p1']['b']), row(pr['pbn']['g']),
            row(pr['pbn']['b']), pr['p2']['W'], row(pr['p2']['b']),
            row(pr['wbn1']['g']), row(pr['wbn1']['b']),
            pr['w1']['W'], row(pr['w1']['b']),
            row(pr['wbn2']['g']), row(pr['wbn2']['b']),
            pr['w2']['W'], row(pr['w2']['b']),
            row(bn2['g']), row(bn2['b']))
    specs = [
        pl.BlockSpec((rb, c), lambda i: (i, 0)),
        pl.BlockSpec((rb * k, c), lambda i: (i, 0)),
        pl.BlockSpec((rb * k, c), lambda i: (i, 0)),
        pl.BlockSpec((rb * k, 3), lambda i: (i, 0)),
        full((3, 3)), full((1, 3)), full((1, 3)), full((1, 3)),
        full((3, c)), full((1, c)),
        full((1, c)), full((1, c)),
        full((c, c8)), full((1, c8)),
        full((1, c8)), full((1, c8)),
        full((c8, c8)), full((1, c8)),
        full((1, c)), full((1, c)),
    ]
    o = pl.pallas_call(
        functools.partial(_attn_kernel, k=k, c=c, share=share),
        grid=(rows // rb,),
        in_specs=specs,
        out_specs=pl.BlockSpec((rb, c), lambda i: (i, 0)),
        out_shape=jax.ShapeDtypeStruct((rows, c), jnp.float32),
    )(*args)
    return o.reshape(bsz, n, c)


def _glm_kernel(x_ref, w_ref, g_ref, bt_ref, o_ref, *, ns):
    mb = o_ref.shape[0]
    y = _dot(x_ref[...], w_ref[...])
    y = jnp.maximum(y * g_ref[...] + bt_ref[...], 0.0)
    o_ref[...] = jnp.max(y.reshape(mb, ns, y.shape[-1]), axis=1)


def _glm_call(g, w, bn, ns):
    bsz, m, _, cin = g.shape
    cout = w.shape[-1]
    rows = bsz * m
    x2 = g.reshape(rows * ns, cin)
    mb = min(rows, max(8, _pow2_floor((1 << 19) // (ns * max(cin, cout)))))
    full = lambda shape: pl.BlockSpec(shape, lambda i: (0, 0))
    y = pl.pallas_call(
        functools.partial(_glm_kernel, ns=ns),
        grid=(rows // mb,),
        in_specs=[
            pl.BlockSpec((mb * ns, cin), lambda i: (i, 0)),
            full((cin, cout)),
            full((1, cout)), full((1, cout)),
        ],
        out_specs=pl.BlockSpec((mb, cout), lambda i: (i, 0)),
        out_shape=jax.ShapeDtypeStruct((rows, cout), jnp.float32),
    )(x2, w, bn['g'].reshape(1, cout), bn['b'].reshape(1, cout))
    return y.reshape(bsz, m, cout)


def _interp_kernel(a_ref, f_ref, d_ref, o_ref):
    r = 1.0 / (d_ref[...] + 1e-8)
    w = r / jnp.sum(r, axis=1, keepdims=True)
    o_ref[...] = a_ref[...] + jnp.sum(f_ref[...] * w[:, :, None], axis=1)


def _interp_call(a, f3, d):
    bsz, n, c = a.shape
    rows = bsz * n
    rb = min(rows, 1024)
    y = pl.pallas_call(
        _interp_kernel,
        grid=(rows // rb,),
        in_specs=[
            pl.BlockSpec((rb, c), lambda i: (i, 0)),
            pl.BlockSpec((rb, 3, c), lambda i: (i, 0, 0)),
            pl.BlockSpec((rb, 3), lambda i: (i, 0)),
        ],
        out_specs=pl.BlockSpec((rb, c), lambda i: (i, 0)),
        out_shape=jax.ShapeDtypeStruct((rows, c), jnp.float32),
    )(a.reshape(rows, c), f3.reshape(rows, 3, c), d.reshape(rows, 3))
    return y.reshape(bsz, n, c)


def _head_kernel(x_ref, w_ref, b_ref, o_ref, *, n):
    mean = jnp.sum(x_ref[0], axis=0, keepdims=True) / n
    o_ref[0] = jnp.maximum(_dot(mean, w_ref[...]) + b_ref[...], 0.0)


def _head_call(x, w, b):
    bsz, n, c = x.shape
    return pl.pallas_call(
        functools.partial(_head_kernel, n=n),
        grid=(bsz,),
        in_specs=[
            pl.BlockSpec((1, n, c), lambda b: (b, 0, 0)),
            pl.BlockSpec((c, c), lambda b: (0, 0)),
            pl.BlockSpec((1, c), lambda b: (0, 0)),
        ],
        out_specs=pl.BlockSpec((1, 1, c), lambda b: (b, 0, 0)),
        out_shape=jax.ShapeDtypeStruct((bsz, 1, c), jnp.float32),
    )(x, w, b.reshape(1, c))


def _take_rows(feat, idx):
    return jax.vmap(lambda f, i: f[i])(feat, idx)


def _block_apply(pr, share, ns, pp, x, knn_idx):
    c = x.shape[-1]
    idn = x
    y = _lin_call(x, pr['l1']['W'], None, pr['bn1'], relu=True)
    ptl = pr['ptl']
    xq = _lin_call(y, ptl['q']['W'], ptl['q']['b'])
    xk = _lin_call(y, ptl['k']['W'], ptl['k']['b'])
    xv = _lin_call(y, ptl['v']['W'], ptl['v']['b'])
    rel = _take_rows(pp, knn_idx) - pp[:, :, None, :]
    gk = _take_rows(xk, knn_idx)
    gv = _take_rows(xv, knn_idx)
    att = _attn_call(xq, gk, gv, rel, ptl, pr['bn2'], share, ns)
    return _lin_call(att, pr['l3']['W'], None, pr['bn3'], relu=True, res=idn)


def _td_down(pr, ns, pp, x):
    bsz, n, c = x.shape
    m = n // 4
    fidx = _fps_call(pp, m)
    np_ = _take_rows(pp, fidx)
    idx, _ = _knn_call(np_, pp, ns)
    gp = _take_rows(pp, idx) - np_[:, :, None, :]
    gx = _take_rows(x, idx)
    g = jnp.concatenate([gp, gx], -1)
    y = _glm_call(g, pr['lin']['W'], pr['bn'], ns)
    return np_, y


def _tu_apply(pr, p1, x1, p2, x2):
    a = _lin_call(x1, pr['l1']['W'], pr['l1']['b'], pr['bn1'], relu=True)
    bf = _lin_call(x2, pr['l2']['W'], pr['l2']['b'], pr['bn2'], relu=True)
    idx, dst = _knn_call(p1, p2, 3)
    f3 = _take_rows(bf, idx)
    return _interp_call(a, f3, dst)


def _tu_head(pr, x):
    bsz, n, c = x.shape
    t = _head_call(x, pr['l2']['W'], pr['l2']['b'])
    y = jnp.concatenate([x, jnp.broadcast_to(t, (bsz, n, c))], -1)
    return _lin_call(y, pr['l1']['W'], None, pr['bn1'], relu=True)


def kernel(p, x, params):
    x0 = jnp.concatenate([p, x], -1)
    h1 = _lin_call(x0, params['enc1']['td']['lin']['W'], None,
                   params['enc1']['td']['bn'], relu=True)
    knn1, _ = _knn_call(p, p, 8)
    h1 = _block_apply(params['enc1']['b0'], 8, 8, p, h1, knn1)
    p2, h2 = _td_down(params['enc2']['td'], 16, p, h1)
    knn2, _ = _knn_call(p2, p2, 16)
    h2 = _block_apply(params['enc2']['b0'], 8, 16, p2, h2, knn2)
    p3, h3 = _td_down(params['enc3']['td'], 16, p2, h2)
    knn3, _ = _knn_call(p3, p3, 16)
    h3 = _block_apply(params['enc3']['b0'], 8, 16, p3, h3, knn3)
    p4, h4 = _td_down(params['enc4']['td'], 16, p3, h3)
    knn4, _ = _knn_call(p4, p4, 16)
    h4 = _block_apply(params['enc4']['b0'], 8, 16, p4, h4, knn4)
    d4 = _tu_head(params['dec4']['tu'], h4)
    d4 = _block_apply(params['dec4']['b0'], 8, 16, p4, d4, knn4)
    d3 = _tu_apply(params['dec3']['tu'], p3, h3, p4, d4)
    d3 = _block_apply(params['dec3']['b0'], 8, 16, p3, d3, knn3)
    d2 = _tu_apply(params['dec2']['tu'], p2, h2, p3, d3)
    d2 = _block_apply(params['dec2']['b0'], 8, 16, p2, d2, knn2)
    d1 = _tu_apply(params['dec1']['tu'], p1 := p, h1, p2, d2)
    d1 = _block_apply(params['dec1']['b0'], 8, 8, p1, d1, knn1)
    return (d4, d3, d2, d1)


# speedup vs baseline: 11.6714x; 5.1825x over previous
import functools
import math

import jax
import jax.numpy as jnp
from jax import lax
from jax.experimental import pallas as pl
from jax.experimental.pallas import tpu as pltpu
from jax.experimental.pallas import tpu_sc as plsc

_PREC = lax.Precision.DEFAULT


def _dot(a, b):
    return lax.dot_general(a, b, (((a.ndim - 1,), (0,)), ((), ())),
                           precision=_PREC, preferred_element_type=jnp.float32)


def _pow2_floor(v):
    return 1 << (int(v).bit_length() - 1)


@functools.lru_cache(maxsize=None)
def _sc_gather_fn(v_rows, d, m):
    nw = 32
    b_per_w = m // nw
    ch = b_per_w
    while ch * d * 4 > 380_000:
        ch //= 2
    assert ch >= 8 and ch % 8 == 0 and b_per_w % ch == 0
    nch = b_per_w // ch
    mesh = plsc.VectorSubcoreMesh(core_axis_name="c", subcore_axis_name="s")

    @functools.partial(
        pl.kernel, mesh=mesh,
        out_type=jax.ShapeDtypeStruct((m, d), jnp.float32),
        scratch_types=[
            pltpu.VMEM((ch,), jnp.int32),
            pltpu.VMEM((ch, d), jnp.float32),
            pltpu.SemaphoreType.DMA,
        ],
    )
    def k(table_hbm, idx_hbm, out_hbm, idx_v, rows_v, sem):
        wid = lax.axis_index("s") * 2 + lax.axis_index("c")
        for j in range(nch):
            base = wid * b_per_w + j * ch
            pltpu.sync_copy(idx_hbm.at[pl.ds(base, ch)], idx_v)
            pltpu.async_copy(table_hbm.at[idx_v], rows_v, sem).wait()
            pltpu.sync_copy(rows_v, out_hbm.at[pl.ds(base, ch)])

    return k


def _sc_gather(table, idx):
    m = idx.shape[0]
    pad = (-m) % 256
    if pad:
        idx = jnp.concatenate([idx, jnp.zeros((pad,), jnp.int32)])
    out = _sc_gather_fn(table.shape[0], table.shape[1], m + pad)(table, idx)
    return out[:m] if pad else out


def _pad128(p2):
    return jnp.pad(p2, ((0, 0), (0, 125)))


def _lin_kernel(x_ref, w_ref, b_ref, g_ref, bt_ref, o_ref, *, relu):
    y = _dot(x_ref[...], w_ref[...])
    y = (y + b_ref[...]) * g_ref[...] + bt_ref[...]
    if relu:
        y = jnp.maximum(y, 0.0)
    o_ref[...] = y


def _lin_res_kernel(x_ref, w_ref, b_ref, g_ref, bt_ref, r_ref, o_ref, *, relu):
    y = _dot(x_ref[...], w_ref[...])
    y = (y + b_ref[...]) * g_ref[...] + bt_ref[...] + r_ref[...]
    if relu:
        y = jnp.maximum(y, 0.0)
    o_ref[...] = y


def _lin_call(x, w, b=None, bn=None, relu=False, res=None):
    lead = x.shape[:-1]
    cin = x.shape[-1]
    cout = w.shape[-1]
    rows = int(math.prod(lead))
    x2 = x.reshape(rows, cin)
    rb = min(rows, 1024)
    b2 = jnp.zeros((1, cout), jnp.float32) if b is None else b.reshape(1, cout)
    if bn is None:
        g2 = jnp.ones((1, cout), jnp.float32)
        bt2 = jnp.zeros((1, cout), jnp.float32)
    else:
        g2 = bn['g'].reshape(1, cout)
        bt2 = bn['b'].reshape(1, cout)
    full = pl.BlockSpec((1, cout), lambda i: (0, 0))
    wspec = pl.BlockSpec((cin, cout), lambda i: (0, 0))
    rspec = pl.BlockSpec((rb, cout), lambda i: (i, 0))
    xspec = pl.BlockSpec((rb, cin), lambda i: (i, 0))
    if res is None:
        fn = functools.partial(_lin_kernel, relu=relu)
        args = (x2, w, b2, g2, bt2)
        specs = [xspec, wspec, full, full, full]
    else:
        fn = functools.partial(_lin_res_kernel, relu=relu)
        args = (x2, w, b2, g2, bt2, res.reshape(rows, cout))
        specs = [xspec, wspec, full, full, full, rspec]
    y = pl.pallas_call(
        fn,
        grid=(rows // rb,),
        in_specs=specs,
        out_specs=rspec,
        out_shape=jax.ShapeDtypeStruct((rows, cout), jnp.float32),
    )(*args)
    return y.reshape(lead + (cout,))


def _knn_kernel(q_ref, st_ref, idx_ref, dst_ref, *, k, ns):
    q = q_ref[0]
    st = st_ref[0]
    boff = pl.program_id(0) * ns
    qq = jnp.sum(q * q, axis=1, keepdims=True)
    ss = jnp.sum(st * st, axis=0, keepdims=True)
    d = qq + ss - 2.0 * _dot(q, st)
    iota = lax.broadcasted_iota(jnp.int32, d.shape, 1)
    for j in range(k):
        m = jnp.min(d, axis=1, keepdims=True)
        ij = jnp.min(jnp.where(d == m, iota, ns), axis=1, keepdims=True)
        idx_ref[0, :, j:j + 1] = ij + boff
        dst_ref[0, :, j:j + 1] = jnp.maximum(m, 0.0)
        if j + 1 < k:
            d = jnp.where(iota == ij, jnp.inf, d)


def _knn_call(q, s, k):
    bsz, nq, _ = q.shape
    ns = s.shape[1]
    st = jnp.swapaxes(s, 1, 2)
    qb = min(nq, max(64, _pow2_floor((1 << 20) // ns)))
    idx, dst = pl.pallas_call(
        functools.partial(_knn_kernel, k=k, ns=ns),
        grid=(bsz, nq // qb),
        in_specs=[
            pl.BlockSpec((1, qb, 3), lambda b, i: (b, i, 0)),
            pl.BlockSpec((1, 3, ns), lambda b, i: (b, 0, 0)),
        ],
        out_specs=[
            pl.BlockSpec((1, qb, k), lambda b, i: (b, i, 0)),
            pl.BlockSpec((1, qb, k), lambda b, i: (b, i, 0)),
        ],
        out_shape=[
            jax.ShapeDtypeStruct((bsz, nq, k), jnp.int32),
            jax.ShapeDtypeStruct((bsz, nq, k), jnp.float32),
        ],
    )(q, st)
    return idx, dst


def _fps_kernel(p_ref, o_ref, *, bsz, m, n8):
    xyz = [[p_ref[b, c] for c in range(3)] for b in range(bsz)]
    fiota = (lax.broadcasted_iota(jnp.int32, (8, n8), 0) * n8
             + lax.broadcasted_iota(jnp.int32, (8, n8), 1))
    n = 8 * n8

    def coord(v, nxt):
        return jnp.sum(jnp.where(fiota == nxt, v, 0.0))

    last0 = [[coord(xyz[b][c], 0) for c in range(3)] for b in range(bsz)]
    for b in range(bsz):
        o_ref[b, 0] = b * n

    def body(i, st):
        last, dd = st
        new_last = []
        new_dd = []
        for b in range(bsz):
            dx = xyz[b][0] - last[b][0]
            dy = xyz[b][1] - last[b][1]
            dz = xyz[b][2] - last[b][2]
            d = (dx * dx + dy * dy) + dz * dz
            db = jnp.minimum(dd[b], d)
            mx = jnp.max(db)
            nxt = jnp.min(jnp.where(db == mx, fiota, n))
            o_ref[b, i] = nxt + b * n
            new_last.append([coord(xyz[b][c], nxt) for c in range(3)])
            new_dd.append(db)
        return new_last, new_dd

    dd0 = [jnp.full((8, n8), 1e10, jnp.float32) for _ in range(bsz)]
    lax.fori_loop(1, m, body, (last0, dd0))


def _fps_call(p, m):
    bsz, n, _ = p.shape
    n8 = n // 8
    pt = jnp.swapaxes(p, 1, 2).reshape(bsz, 3, 8, n8)
    return pl.pallas_call(
        functools.partial(_fps_kernel, bsz=bsz, m=m, n8=n8),
        in_specs=[pl.BlockSpec(memory_space=pltpu.VMEM)],
        out_specs=pl.BlockSpec(memory_space=pltpu.SMEM),
        out_shape=jax.ShapeDtypeStruct((bsz, m), jnp.int32),
    )(pt)


def _attn_kernel(xq_ref, qp_ref, g_ref,
                 p1w_ref, p1b_ref, pg_ref, pb_ref, p2w_ref, p2b_ref,
                 g1_ref, t1_ref, w1w_ref, w1b_ref,
                 g2_ref, t2_ref, w2w_ref, w2b_ref,
                 bg_ref, bb_ref, o_ref, *, k, c, share):
    rb = o_ref.shape[0]
    rbk = rb * k
    g = g_ref[...]
    xk = g[:, 0:c]
    xv = g[:, c:2 * c]
    gp = g[:, 2 * c:2 * c + 3]
    rel = (gp.reshape(rb, k, 3)
           - qp_ref[...].reshape(rb, 1, 3)).reshape(rbk, 3)
    t = _dot(rel, p1w_ref[...]) + p1b_ref[...]
    t = jnp.maximum(t * pg_ref[...] + pb_ref[...], 0.0)
    pe = _dot(t, p2w_ref[...]) + p2b_ref[...]
    xq = xq_ref[...]
    w3 = (xk.reshape(rb, k, c) - xq.reshape(rb, 1, c)
          + pe.reshape(rb, k, c))
    w = w3.reshape(rbk, c)
    w = jnp.maximum(w * g1_ref[...] + t1_ref[...], 0.0)
    w = _dot(w, w1w_ref[...]) + w1b_ref[...]
    w = jnp.maximum(w * g2_ref[...] + t2_ref[...], 0.0)
    w = _dot(w, w2w_ref[...]) + w2b_ref[...]
    c8 = c // share
    w3 = w.reshape(rb, k, c8)
    w3 = w3 - jnp.max(w3, axis=1, keepdims=True)
    e = jnp.exp(w3)
    sm = e / jnp.sum(e, axis=1, keepdims=True)
    v3 = (xv + pe).reshape(rb, k, c)
    wfull = jnp.concatenate([sm] * share, axis=2)
    o = jnp.sum(v3 * wfull, axis=1)
    o_ref[...] = jnp.maximum(o * bg_ref[...] + bb_ref[...], 0.0)


def _attn_call(xq, qp, g, pr, bn2, share, k):
    bsz, n, c = xq.shape
    rows = bsz * n
    c8 = c // share
    cg = 2 * c + 128
    rb = min(rows, max(64, _pow2_floor((1 << 19) // (k * c))))
    full = lambda shape: pl.BlockSpec(shape, lambda i: (0, 0))

    def row(v):
        return v.reshape(1, -1)

    args = (xq.reshape(rows, c), qp, g,
            pr['p1']['W'], row(pr['p1']['b']), row(pr['pbn']['g']),
            row(pr['pbn']['b']), pr['p2']['W'], row(pr['p2']['b']),
            row(pr['wbn1']['g']), row(pr['wbn1']['b']),
            pr['w1']['W'], row(pr['w1']['b']),
            row(pr['wbn2']['g']), row(pr['wbn2']['b']),
            pr['w2']['W'], row(pr['w2']['b']),
            row(bn2['g']), row(bn2['b']))
    specs = [
        pl.BlockSpec((rb, c), lambda i: (i, 0)),
        pl.BlockSpec((rb, 3), lambda i: (i, 0)),
        pl.BlockSpec((rb * k, cg), lambda i: (i, 0)),
        full((3, 3)), full((1, 3)), full((1, 3)), full((1, 3)),
        full((3, c)), full((1, c)),
        full((1, c)), full((1, c)),
        full((c, c8)), full((1, c8)),
        full((1, c8)), full((1, c8)),
        full((c8, c8)), full((1, c8)),
        full((1, c)), full((1, c)),
    ]
    o = pl.pallas_call(
        functools.partial(_attn_kernel, k=k, c=c, share=share),
        grid=(rows // rb,),
        in_specs=specs,
        out_specs=pl.BlockSpec((rb, c), lambda i: (i, 0)),
        out_shape=jax.ShapeDtypeStruct((rows, c), jnp.float32),
    )(*args)
    return o.reshape(bsz, n, c)


def _glm_kernel(g_ref, qp_ref, w_ref, gg_ref, bt_ref, o_ref, *, ns, cin):
    mb = o_ref.shape[0]
    g = g_ref[...]
    rel = (g[:, cin:cin + 3].reshape(mb, ns, 3)
           - qp_ref[...].reshape(mb, 1, 3)).reshape(mb * ns, 3)
    x = jnp.concatenate([rel, g[:, 0:cin]], axis=1)
    y = _dot(x, w_ref[...])
    y = jnp.maximum(y * gg_ref[...] + bt_ref[...], 0.0)
    o_ref[...] = jnp.max(y.reshape(mb, ns, y.shape[-1]), axis=1)


def _glm_call(g, qp, w, bn, ns, cin):
    rows = qp.shape[0]
    cout = w.shape[-1]
    gw = g.shape[-1]
    mb = min(rows, max(8, _pow2_floor((1 << 19) // (ns * max(gw, cout)))))
    full = lambda shape: pl.BlockSpec(shape, lambda i: (0, 0))
    y = pl.pallas_call(
        functools.partial(_glm_kernel, ns=ns, cin=cin),
        grid=(rows // mb,),
        in_specs=[
            pl.BlockSpec((mb * ns, gw), lambda i: (i, 0)),
            pl.BlockSpec((mb, 3), lambda i: (i, 0)),
            full((cin + 3, cout)),
            full((1, cout)), full((1, cout)),
        ],
        out_specs=pl.BlockSpec((mb, cout), lambda i: (i, 0)),
        out_shape=jax.ShapeDtypeStruct((rows, cout), jnp.float32),
    )(g, qp, w, bn['g'].reshape(1, cout), bn['b'].reshape(1, cout))
    return y


def _interp_kernel(a_ref, f_ref, d_ref, o_ref, *, c):
    r = 1.0 / (d_ref[...] + 1e-8)
    w = r / jnp.sum(r, axis=1, keepdims=True)
    f = f_ref[...][:, :, 0:c]
    o_ref[...] = a_ref[...] + jnp.sum(f * w[:, :, None], axis=1)


def _interp_call(a, f3, d):
    bsz, n, c = a.shape
    cg = f3.shape[-1]
    rows = bsz * n
    rb = min(rows, 1024)
    y = pl.pallas_call(
        functools.partial(_interp_kernel, c=c),
        grid=(rows // rb,),
        in_specs=[
            pl.BlockSpec((rb, c), lambda i: (i, 0)),
            pl.BlockSpec((rb, 3, cg), lambda i: (i, 0, 0)),
            pl.BlockSpec((rb, 3), lambda i: (i, 0)),
        ],
        out_specs=pl.BlockSpec((rb, c), lambda i: (i, 0)),
        out_shape=jax.ShapeDtypeStruct((rows, c), jnp.float32),
    )(a.reshape(rows, c), f3.reshape(rows, 3, cg), d.reshape(rows, 3))
    return y.reshape(bsz, n, c)


def _head_kernel(x_ref, w_ref, b_ref, o_ref, *, n):
    mean = jnp.sum(x_ref[0], axis=0, keepdims=True) / n
    o_ref[0] = jnp.maximum(_dot(mean, w_ref[...]) + b_ref[...], 0.0)


def _head_call(x, w, b):
    bsz, n, c = x.shape
    return pl.pallas_call(
        functools.partial(_head_kernel, n=n),
        grid=(bsz,),
        in_specs=[
            pl.BlockSpec((1, n, c), lambda b: (b, 0, 0)),
            pl.BlockSpec((c, c), lambda b: (0, 0)),
            pl.BlockSpec((1, c), lambda b: (0, 0)),
        ],
        out_specs=pl.BlockSpec((1, 1, c), lambda b: (b, 0, 0)),
        out_shape=jax.ShapeDtypeStruct((bsz, 1, c), jnp.float32),
    )(x, w, b.reshape(1, c))


def _block_apply(pr, share, ns, pp128, qp, x, knn_idx):
    bsz, n, c = x.shape
    rows = bsz * n
    idn = x
    y = _lin_call(x, pr['l1']['W'], None, pr['bn1'], relu=True)
    ptl = pr['ptl']
    xq = _lin_call(y, ptl['q']['W'], ptl['q']['b'])
    wkv = jnp.concatenate([ptl['k']['W'], ptl['v']['W']], axis=1)
    bkv = jnp.concatenate([ptl['k']['b'], ptl['v']['b']])
    kv = _lin_call(y, wkv, bkv)
    table = jnp.concatenate([kv.reshape(rows, 2 * c), pp128], axis=1)
    g = _sc_gather(table, knn_idx.reshape(-1))
    att = _attn_call(xq, qp, g, ptl, pr['bn2'], share, ns)
    return _lin_call(att, pr['l3']['W'], None, pr['bn3'], relu=True, res=idn)


def _td_down(pr, ns, pp, pp128, x):
    bsz, n, c = x.shape
    m = n // 4
    fidx = _fps_call(pp, m)
    np_rows = _sc_gather(pp128, fidx.reshape(-1))[:, :3]
    np_ = np_rows.reshape(bsz, m, 3)
    idx, _ = _knn_call(np_, pp, ns)
    gw = -(-(c + 3) // 128) * 128
    table = jnp.pad(
        jnp.concatenate([x.reshape(bsz * n, c), pp.reshape(bsz * n, 3)], axis=1),
        ((0, 0), (0, gw - c - 3)))
    g = _sc_gather(table, idx.reshape(-1))
    y = _glm_call(g, np_rows, pr['lin']['W'], pr['bn'], ns, c)
    return np_, y.reshape(bsz, m, -1)


def _tu_apply(pr, p1, x1, p2, x2):
    bsz, n1 = p1.shape[:2]
    a = _lin_call(x1, pr['l1']['W'], pr['l1']['b'], pr['bn1'], relu=True)
    w2, b2 = pr['l2']['W'], pr['l2']['b']
    bn2 = pr['bn2']
    c = w2.shape[-1]
    if c < 128:
        pad = 128 - c
        w2 = jnp.pad(w2, ((0, 0), (0, pad)))
        b2 = jnp.pad(b2, (0, pad))
        bn2 = {'g': jnp.pad(bn2['g'], (0, pad)), 'b': jnp.pad(bn2['b'], (0, pad))}
    bf = _lin_call(x2, w2, b2, bn2, relu=True)
    idx, dst = _knn_call(p1, p2, 3)
    cg = bf.shape[-1]
    f = _sc_gather(bf.reshape(-1, cg), idx.reshape(-1))
    return _interp_call(a, f.reshape(bsz, n1, 3, cg), dst)


def _tu_head(pr, x):
    bsz, n, c = x.shape
    t = _head_call(x, pr['l2']['W'], pr['l2']['b'])
    y = jnp.concatenate([x, jnp.broadcast_to(t, (bsz, n, c))], -1)
    return _lin_call(y, pr['l1']['W'], None, pr['bn1'], relu=True)


def kernel(p, x, params):
    bsz, n, _ = p.shape
    x0 = jnp.concatenate([p, x], -1)
    p1 = p
    p1f = p1.reshape(bsz * n, 3)
    p116 = _pad128(p1f)
    h1 = _lin_call(x0, params['enc1']['td']['lin']['W'], None,
                   params['enc1']['td']['bn'], relu=True)
    knn1, _ = _knn_call(p1, p1, 8)
    h1 = _block_apply(params['enc1']['b0'], 8, 8, p116, p1f, h1, knn1)
    p2, h2 = _td_down(params['enc2']['td'], 16, p1, p116, h1)
    p2f = p2.reshape(-1, 3)
    p216 = _pad128(p2f)
    knn2, _ = _knn_call(p2, p2, 16)
    h2 = _block_apply(params['enc2']['b0'], 8, 16, p216, p2f, h2, knn2)
    p3, h3 = _td_down(params['enc3']['td'], 16, p2, p216, h2)
    p3f = p3.reshape(-1, 3)
    p316 = _pad128(p3f)
    knn3, _ = _knn_call(p3, p3, 16)
    h3 = _block_apply(params['enc3']['b0'], 8, 16, p316, p3f, h3, knn3)
    p4, h4 = _td_down(params['enc4']['td'], 16, p3, p316, h3)
    p4f = p4.reshape(-1, 3)
    p416 = _pad128(p4f)
    knn4, _ = _knn_call(p4, p4, 16)
    h4 = _block_apply(params['enc4']['b0'], 8, 16, p416, p4f, h4, knn4)
    d4 = _tu_head(params['dec4']['tu'], h4)
    d4 = _block_apply(params['dec4']['b0'], 8, 16, p416, p4f, d4, knn4)
    d3 = _tu_apply(params['dec3']['tu'], p3, h3, p4, d4)
    d3 = _block_apply(params['dec3']['b0'], 8, 16, p316, p3f, d3, knn3)
    d2 = _tu_apply(params['dec2']['tu'], p2, h2, p3, d3)
    d2 = _block_apply(params['dec2']['b0'], 8, 16, p216, p2f, d2, knn2)
    d1 = _tu_apply(params['dec1']['tu'], p1, h1, p2, d2)
    d1 = _block_apply(params['dec1']['b0'], 8, 8, p116, p1f, d1, knn1)
    return (d4, d3, d2, d1)


# speedup vs baseline: 15.1370x; 1.2969x over previous
import functools
import math

import jax
import jax.numpy as jnp
from jax import lax
from jax.experimental import pallas as pl
from jax.experimental.pallas import tpu as pltpu
from jax.experimental.pallas import tpu_sc as plsc

_PREC = lax.Precision.DEFAULT


def _dot(a, b):
    return lax.dot_general(a, b, (((a.ndim - 1,), (0,)), ((), ())),
                           precision=_PREC, preferred_element_type=jnp.float32)


def _pow2_floor(v):
    return 1 << (int(v).bit_length() - 1)


@functools.lru_cache(maxsize=None)
def _sc_gather_fn(v_rows, d, m):
    nw = 32
    b_per_w = m // nw
    ch = b_per_w
    while ch * d * 4 > 200_000:
        ch //= 2
    if ch == b_per_w and ch >= 16:
        ch //= 2
    assert ch >= 8 and ch % 8 == 0 and b_per_w % ch == 0
    nch = b_per_w // ch
    mesh = plsc.VectorSubcoreMesh(core_axis_name="c", subcore_axis_name="s")

    @functools.partial(
        pl.kernel, mesh=mesh,
        out_type=jax.ShapeDtypeStruct((m, d), jnp.float32),
        scratch_types=[
            pltpu.VMEM((ch,), jnp.int32),
            pltpu.VMEM((ch,), jnp.int32),
            pltpu.VMEM((ch, d), jnp.float32),
            pltpu.VMEM((ch, d), jnp.float32),
            pltpu.SemaphoreType.DMA,
            pltpu.SemaphoreType.DMA,
            pltpu.SemaphoreType.DMA,
            pltpu.SemaphoreType.DMA,
        ],
    )
    def k(table_hbm, idx_hbm, out_hbm, iv0, iv1, r0, r1, g0, g1, w0, w1):
        ivs, rows = (iv0, iv1), (r0, r1)
        gs, ws = (g0, g1), (w0, w1)
        wid = lax.axis_index("s") * 2 + lax.axis_index("c")
        gh, wh = {}, {}

        def start_gather(j):
            base = wid * b_per_w + j * ch
            pltpu.sync_copy(idx_hbm.at[pl.ds(base, ch)], ivs[j % 2])
            gh[j] = pltpu.async_copy(table_hbm.at[ivs[j % 2]], rows[j % 2],
                                     gs[j % 2])

        def start_wb(j):
            base = wid * b_per_w + j * ch
            wh[j] = pltpu.async_copy(rows[j % 2], out_hbm.at[pl.ds(base, ch)],
                                     ws[j % 2])

        start_gather(0)
        for j in range(nch):
            if j + 1 < nch:
                if j >= 1:
                    wh[j - 1].wait()
                start_gather(j + 1)
            gh[j].wait()
            start_wb(j)
        wh[nch - 1].wait()
        if nch >= 2:
            wh[nch - 2].wait()

    return k


def _sc_gather(table, idx):
    m = idx.shape[0]
    pad = (-m) % 256
    if pad:
        idx = jnp.concatenate([idx, jnp.zeros((pad,), jnp.int32)])
    out = _sc_gather_fn(table.shape[0], table.shape[1], m + pad)(table, idx)
    return out[:m] if pad else out


def _pad128(p2):
    return jnp.pad(p2, ((0, 0), (0, 125)))


def _lin_kernel(x_ref, w_ref, b_ref, g_ref, bt_ref, o_ref, *, relu):
    y = _dot(x_ref[...], w_ref[...])
    y = (y + b_ref[...]) * g_ref[...] + bt_ref[...]
    if relu:
        y = jnp.maximum(y, 0.0)
    o_ref[...] = y


def _lin_res_kernel(x_ref, w_ref, b_ref, g_ref, bt_ref, r_ref, o_ref, *, relu):
    y = _dot(x_ref[...], w_ref[...])
    y = (y + b_ref[...]) * g_ref[...] + bt_ref[...] + r_ref[...]
    if relu:
        y = jnp.maximum(y, 0.0)
    o_ref[...] = y


def _lin_call(x, w, b=None, bn=None, relu=False, res=None):
    lead = x.shape[:-1]
    cin = x.shape[-1]
    cout = w.shape[-1]
    rows = int(math.prod(lead))
    x2 = x.reshape(rows, cin)
    rb = min(rows, 1024)
    b2 = jnp.zeros((1, cout), jnp.float32) if b is None else b.reshape(1, cout)
    if bn is None:
        g2 = jnp.ones((1, cout), jnp.float32)
        bt2 = jnp.zeros((1, cout), jnp.float32)
    else:
        g2 = bn['g'].reshape(1, cout)
        bt2 = bn['b'].reshape(1, cout)
    full = pl.BlockSpec((1, cout), lambda i: (0, 0))
    wspec = pl.BlockSpec((cin, cout), lambda i: (0, 0))
    rspec = pl.BlockSpec((rb, cout), lambda i: (i, 0))
    xspec = pl.BlockSpec((rb, cin), lambda i: (i, 0))
    if res is None:
        fn = functools.partial(_lin_kernel, relu=relu)
        args = (x2, w, b2, g2, bt2)
        specs = [xspec, wspec, full, full, full]
    else:
        fn = functools.partial(_lin_res_kernel, relu=relu)
        args = (x2, w, b2, g2, bt2, res.reshape(rows, cout))
        specs = [xspec, wspec, full, full, full, rspec]
    y = pl.pallas_call(
        fn,
        grid=(rows // rb,),
        in_specs=specs,
        out_specs=rspec,
        out_shape=jax.ShapeDtypeStruct((rows, cout), jnp.float32),
    )(*args)
    return y.reshape(lead + (cout,))


def _knn_kernel(q_ref, st_ref, idx_ref, dst_ref, *, k, ns):
    q = q_ref[0]
    st = st_ref[0]
    boff = pl.program_id(0) * ns
    qq = jnp.sum(q * q, axis=1, keepdims=True)
    ss = jnp.sum(st * st, axis=0, keepdims=True)
    d = qq + ss - 2.0 * _dot(q, st)
    iota = lax.broadcasted_iota(jnp.int32, d.shape, 1)
    for j in range(k):
        m = jnp.min(d, axis=1, keepdims=True)
        ij = jnp.min(jnp.where(d == m, iota, ns), axis=1, keepdims=True)
        idx_ref[0, :, j:j + 1] = ij + boff
        dst_ref[0, :, j:j + 1] = jnp.maximum(m, 0.0)
        if j + 1 < k:
            d = jnp.where(iota == ij, jnp.inf, d)


def _knn_call(q, s, k):
    bsz, nq, _ = q.shape
    ns = s.shape[1]
    st = jnp.swapaxes(s, 1, 2)
    qb = min(nq, max(64, _pow2_floor((1 << 20) // ns)))
    idx, dst = pl.pallas_call(
        functools.partial(_knn_kernel, k=k, ns=ns),
        grid=(bsz, nq // qb),
        in_specs=[
            pl.BlockSpec((1, qb, 3), lambda b, i: (b, i, 0)),
            pl.BlockSpec((1, 3, ns), lambda b, i: (b, 0, 0)),
        ],
        out_specs=[
            pl.BlockSpec((1, qb, k), lambda b, i: (b, i, 0)),
            pl.BlockSpec((1, qb, k), lambda b, i: (b, i, 0)),
        ],
        out_shape=[
            jax.ShapeDtypeStruct((bsz, nq, k), jnp.int32),
            jax.ShapeDtypeStruct((bsz, nq, k), jnp.float32),
        ],
    )(q, st)
    return idx, dst


def _fps_kernel(p_ref, o_ref, *, bsz, m, n8):
    xyz = [[p_ref[b, c] for c in range(3)] for b in range(bsz)]
    fiota = (lax.broadcasted_iota(jnp.int32, (8, n8), 0) * n8
             + lax.broadcasted_iota(jnp.int32, (8, n8), 1))
    n = 8 * n8

    def red11(v, op):
        return op(op(v, axis=1, keepdims=True), axis=0, keepdims=True)

    def coord(v, nxt):
        return red11(jnp.where(fiota == nxt, v, 0.0), jnp.sum)

    zero = jnp.zeros((1, 1), jnp.int32)
    last0 = [[coord(xyz[b][c], zero) for c in range(3)] for b in range(bsz)]
    for b in range(bsz):
        o_ref[0:1, b:b + 1] = zero + b * n

    def body(i, st):
        last, dd = st
        new_last = []
        new_dd = []
        for b in range(bsz):
            dx = xyz[b][0] - last[b][0]
            dy = xyz[b][1] - last[b][1]
            dz = xyz[b][2] - last[b][2]
            d = (dx * dx + dy * dy) + dz * dz
            db = jnp.minimum(dd[b], d)
            mx = red11(db, jnp.max)
            nxt = red11(jnp.where(db == mx, fiota, n), jnp.min)
            o_ref[pl.ds(i, 1), b:b + 1] = nxt + b * n
            new_last.append([coord(xyz[b][c], nxt) for c in range(3)])
            new_dd.append(db)
        return new_last, new_dd

    dd0 = [jnp.full((8, n8), 1e10, jnp.float32) for _ in range(bsz)]
    lax.fori_loop(1, m, body, (last0, dd0))


def _fps_call(p, m):
    bsz, n, _ = p.shape
    n8 = n // 8
    pt = jnp.swapaxes(p, 1, 2).reshape(bsz, 3, 8, n8)
    out = pl.pallas_call(
        functools.partial(_fps_kernel, bsz=bsz, m=m, n8=n8),
        in_specs=[pl.BlockSpec(memory_space=pltpu.VMEM)],
        out_specs=pl.BlockSpec(memory_space=pltpu.VMEM),
        out_shape=jax.ShapeDtypeStruct((m, bsz), jnp.int32),
    )(pt)
    return out.T


def _attn_kernel(xq_ref, qp_ref, g_ref,
                 p1w_ref, p1b_ref, pg_ref, pb_ref, p2w_ref, p2b_ref,
                 g1_ref, t1_ref, w1w_ref, w1b_ref,
                 g2_ref, t2_ref, w2w_ref, w2b_ref,
                 bg_ref, bb_ref, o_ref, *, k, c, share):
    rb = o_ref.shape[0]
    rbk = rb * k
    g = g_ref[...]
    xk = g[:, 0:c]
    xv = g[:, c:2 * c]
    gp = g[:, 2 * c:2 * c + 3]
    rel = (gp.reshape(rb, k, 3)
           - qp_ref[...].reshape(rb, 1, 3)).reshape(rbk, 3)
    t = _dot(rel, p1w_ref[...]) + p1b_ref[...]
    t = jnp.maximum(t * pg_ref[...] + pb_ref[...], 0.0)
    pe = _dot(t, p2w_ref[...]) + p2b_ref[...]
    xq = xq_ref[...]
    w3 = (xk.reshape(rb, k, c) - xq.reshape(rb, 1, c)
          + pe.reshape(rb, k, c))
    w = w3.reshape(rbk, c)
    w = jnp.maximum(w * g1_ref[...] + t1_ref[...], 0.0)
    w = _dot(w, w1w_ref[...]) + w1b_ref[...]
    w = jnp.maximum(w * g2_ref[...] + t2_ref[...], 0.0)
    w = _dot(w, w2w_ref[...]) + w2b_ref[...]
    c8 = c // share
    w3 = w.reshape(rb, k, c8)
    w3 = w3 - jnp.max(w3, axis=1, keepdims=True)
    e = jnp.exp(w3)
    sm = e / jnp.sum(e, axis=1, keepdims=True)
    v3 = (xv + pe).reshape(rb, k, c)
    wfull = jnp.concatenate([sm] * share, axis=2)
    o = jnp.sum(v3 * wfull, axis=1)
    o_ref[...] = jnp.maximum(o * bg_ref[...] + bb_ref[...], 0.0)


def _attn_call(xq, qp, g, pr, bn2, share, k):
    bsz, n, c = xq.shape
    rows = bsz * n
    c8 = c // share
    cg = 2 * c + 128
    rb = min(rows, max(64, _pow2_floor((1 << 19) // (k * c))))
    full = lambda shape: pl.BlockSpec(shape, lambda i: (0, 0))

    def row(v):
        return v.reshape(1, -1)

    args = (xq.reshape(rows, c), qp, g,
            pr['p1']['W'], row(pr['p1']['b']), row(pr['pbn']['g']),
            row(pr['pbn']['b']), pr['p2']['W'], row(pr['p2']['b']),
            row(pr['wbn1']['g']), row(pr['wbn1']['b']),
            pr['w1']['W'], row(pr['w1']['b']),
            row(pr['wbn2']['g']), row(pr['wbn2']['b']),
            pr['w2']['W'], row(pr['w2']['b']),
            row(bn2['g']), row(bn2['b']))
    specs = [
        pl.BlockSpec((rb, c), lambda i: (i, 0)),
        pl.BlockSpec((rb, 3), lambda i: (i, 0)),
        pl.BlockSpec((rb * k, cg), lambda i: (i, 0)),
        full((3, 3)), full((1, 3)), full((1, 3)), full((1, 3)),
        full((3, c)), full((1, c)),
        full((1, c)), full((1, c)),
        full((c, c8)), full((1, c8)),
        full((1, c8)), full((1, c8)),
        full((c8, c8)), full((1, c8)),
        full((1, c)), full((1, c)),
    ]
    o = pl.pallas_call(
        functools.partial(_attn_kernel, k=k, c=c, share=share),
        grid=(rows // rb,),
        in_specs=specs,
        out_specs=pl.BlockSpec((rb, c), lambda i: (i, 0)),
        out_shape=jax.ShapeDtypeStruct((rows, c), jnp.float32),
    )(*args)
    return o.reshape(bsz, n, c)


def _glm_kernel(g_ref, qp_ref, w_ref, gg_ref, bt_ref, o_ref, *, ns, cin):
    mb = o_ref.shape[0]
    g = g_ref[...]
    rel = (g[:, cin:cin + 3].reshape(mb, ns, 3)
           - qp_ref[...].reshape(mb, 1, 3)).reshape(mb * ns, 3)
    x = jnp.concatenate([rel, g[:, 0:cin]], axis=1)
    y = _dot(x, w_ref[...])
    y = jnp.maximum(y * gg_ref[...] + bt_ref[...], 0.0)
    o_ref[...] = jnp.max(y.reshape(mb, ns, y.shape[-1]), axis=1)


def _glm_call(g, qp, w, bn, ns, cin):
    rows = qp.shape[0]
    cout = w.shape[-1]
    gw = g.shape[-1]
    mb = min(rows, max(8, _pow2_floor((1 << 19) // (ns * max(gw, cout)))))
    full = lambda shape: pl.BlockSpec(shape, lambda i: (0, 0))
    y = pl.pallas_call(
        functools.partial(_glm_kernel, ns=ns, cin=cin),
        grid=(rows // mb,),
        in_specs=[
            pl.BlockSpec((mb * ns, gw), lambda i: (i, 0)),
            pl.BlockSpec((mb, 3), lambda i: (i, 0)),
            full((cin + 3, cout)),
            full((1, cout)), full((1, cout)),
        ],
        out_specs=pl.BlockSpec((mb, cout), lambda i: (i, 0)),
        out_shape=jax.ShapeDtypeStruct((rows, cout), jnp.float32),
    )(g, qp, w, bn['g'].reshape(1, cout), bn['b'].reshape(1, cout))
    return y


def _interp_kernel(a_ref, f_ref, d_ref, o_ref, *, c):
    r = 1.0 / (d_ref[...] + 1e-8)
    w = r / jnp.sum(r, axis=1, keepdims=True)
    f = f_ref[...][:, :, 0:c]
    o_ref[...] = a_ref[...] + jnp.sum(f * w[:, :, None], axis=1)


def _interp_call(a, f3, d):
    bsz, n, c = a.shape
    cg = f3.shape[-1]
    rows = bsz * n
    rb = min(rows, 1024)
    y = pl.pallas_call(
        functools.partial(_interp_kernel, c=c),
        grid=(rows // rb,),
        in_specs=[
            pl.BlockSpec((rb, c), lambda i: (i, 0)),
            pl.BlockSpec((rb, 3, cg), lambda i: (i, 0, 0)),
            pl.BlockSpec((rb, 3), lambda i: (i, 0)),
        ],
        out_specs=pl.BlockSpec((rb, c), lambda i: (i, 0)),
        out_shape=jax.ShapeDtypeStruct((rows, c), jnp.float32),
    )(a.reshape(rows, c), f3.reshape(rows, 3, cg), d.reshape(rows, 3))
    return y.reshape(bsz, n, c)


def _head_kernel(x_ref, w_ref, b_ref, o_ref, *, n):
    mean = jnp.sum(x_ref[0], axis=0, keepdims=True) / n
    o_ref[0] = jnp.maximum(_dot(mean, w_ref[...]) + b_ref[...], 0.0)


def _head_call(x, w, b):
    bsz, n, c = x.shape
    return pl.pallas_call(
        functools.partial(_head_kernel, n=n),
        grid=(bsz,),
        in_specs=[
            pl.BlockSpec((1, n, c), lambda b: (b, 0, 0)),
            pl.BlockSpec((c, c), lambda b: (0, 0)),
            pl.BlockSpec((1, c), lambda b: (0, 0)),
        ],
        out_specs=pl.BlockSpec((1, 1, c), lambda b: (b, 0, 0)),
        out_shape=jax.ShapeDtypeStruct((bsz, 1, c), jnp.float32),
    )(x, w, b.reshape(1, c))


def _block_apply(pr, share, ns, pp128, qp, x, knn_idx):
    bsz, n, c = x.shape
    rows = bsz * n
    idn = x
    y = _lin_call(x, pr['l1']['W'], None, pr['bn1'], relu=True)
    ptl = pr['ptl']
    xq = _lin_call(y, ptl['q']['W'], ptl['q']['b'])
    wkv = jnp.concatenate([ptl['k']['W'], ptl['v']['W']], axis=1)
    bkv = jnp.concatenate([ptl['k']['b'], ptl['v']['b']])
    kv = _lin_call(y, wkv, bkv)
    table = jnp.concatenate([kv.reshape(rows, 2 * c), pp128], axis=1)
    g = _sc_gather(table, knn_idx.reshape(-1))
    att = _attn_call(xq, qp, g, ptl, pr['bn2'], share, ns)
    return _lin_call(att, pr['l3']['W'], None, pr['bn3'], relu=True, res=idn)


def _td_down(pr, ns, pp, pp128, x):
    bsz, n, c = x.shape
    m = n // 4
    fidx = _fps_call(pp, m)
    np_rows = _sc_gather(pp128, fidx.reshape(-1))[:, :3]
    np_ = np_rows.reshape(bsz, m, 3)
    idx, _ = _knn_call(np_, pp, ns)
    gw = -(-(c + 3) // 128) * 128
    table = jnp.pad(
        jnp.concatenate([x.reshape(bsz * n, c), pp.reshape(bsz * n, 3)], axis=1),
        ((0, 0), (0, gw - c - 3)))
    g = _sc_gather(table, idx.reshape(-1))
    y = _glm_call(g, np_rows, pr['lin']['W'], pr['bn'], ns, c)
    return np_, y.reshape(bsz, m, -1)


def _tu_apply(pr, p1, x1, p2, x2):
    bsz, n1 = p1.shape[:2]
    a = _lin_call(x1, pr['l1']['W'], pr['l1']['b'], pr['bn1'], relu=True)
    w2, b2 = pr['l2']['W'], pr['l2']['b']
    bn2 = pr['bn2']
    c = w2.shape[-1]
    if c < 128:
        pad = 128 - c
        w2 = jnp.pad(w2, ((0, 0), (0, pad)))
        b2 = jnp.pad(b2, (0, pad))
        bn2 = {'g': jnp.pad(bn2['g'], (0, pad)), 'b': jnp.pad(bn2['b'], (0, pad))}
    bf = _lin_call(x2, w2, b2, bn2, relu=True)
    idx, dst = _knn_call(p1, p2, 3)
    cg = bf.shape[-1]
    f = _sc_gather(bf.reshape(-1, cg), idx.reshape(-1))
    return _interp_call(a, f.reshape(bsz, n1, 3, cg), dst)


def _tu_head(pr, x):
    bsz, n, c = x.shape
    t = _head_call(x, pr['l2']['W'], pr['l2']['b'])
    y = jnp.concatenate([x, jnp.broadcast_to(t, (bsz, n, c))], -1)
    return _lin_call(y, pr['l1']['W'], None, pr['bn1'], relu=True)


def kernel(p, x, params):
    bsz, n, _ = p.shape
    x0 = jnp.concatenate([p, x], -1)
    p1 = p
    p1f = p1.reshape(bsz * n, 3)
    p116 = _pad128(p1f)
    h1 = _lin_call(x0, params['enc1']['td']['lin']['W'], None,
                   params['enc1']['td']['bn'], relu=True)
    knn1, _ = _knn_call(p1, p1, 8)
    h1 = _block_apply(params['enc1']['b0'], 8, 8, p116, p1f, h1, knn1)
    p2, h2 = _td_down(params['enc2']['td'], 16, p1, p116, h1)
    p2f = p2.reshape(-1, 3)
    p216 = _pad128(p2f)
    knn2, _ = _knn_call(p2, p2, 16)
    h2 = _block_apply(params['enc2']['b0'], 8, 16, p216, p2f, h2, knn2)
    p3, h3 = _td_down(params['enc3']['td'], 16, p2, p216, h2)
    p3f = p3.reshape(-1, 3)
    p316 = _pad128(p3f)
    knn3, _ = _knn_call(p3, p3, 16)
    h3 = _block_apply(params['enc3']['b0'], 8, 16, p316, p3f, h3, knn3)
    p4, h4 = _td_down(params['enc4']['td'], 16, p3, p316, h3)
    p4f = p4.reshape(-1, 3)
    p416 = _pad128(p4f)
    knn4, _ = _knn_call(p4, p4, 16)
    h4 = _block_apply(params['enc4']['b0'], 8, 16, p416, p4f, h4, knn4)
    d4 = _tu_head(params['dec4']['tu'], h4)
    d4 = _block_apply(params['dec4']['b0'], 8, 16, p416, p4f, d4, knn4)
    d3 = _tu_apply(params['dec3']['tu'], p3, h3, p4, d4)
    d3 = _block_apply(params['dec3']['b0'], 8, 16, p316, p3f, d3, knn3)
    d2 = _tu_apply(params['dec2']['tu'], p2, h2, p3, d3)
    d2 = _block_apply(params['dec2']['b0'], 8, 16, p216, p2f, d2, knn2)
    d1 = _tu_apply(params['dec1']['tu'], p1, h1, p2, d2)
    d1 = _block_apply(params['dec1']['b0'], 8, 8, p116, p1f, d1, knn1)
    return (d4, d3, d2, d1)


# speedup vs baseline: 15.9705x; 1.0551x over previous
import functools
import math

import jax
import jax.numpy as jnp
from jax import lax
from jax.experimental import pallas as pl
from jax.experimental.pallas import tpu as pltpu
from jax.experimental.pallas import tpu_sc as plsc

_PREC = lax.Precision.DEFAULT


def _dot(a, b):
    return lax.dot_general(a, b, (((a.ndim - 1,), (0,)), ((), ())),
                           precision=_PREC, preferred_element_type=jnp.float32)


def _pow2_floor(v):
    return 1 << (int(v).bit_length() - 1)


@functools.lru_cache(maxsize=None)
def _sc_gather_fn(v_rows, d, m):
    nw = 32
    b_per_w = m // nw
    ch = b_per_w
    while ch * d * 4 > 200_000:
        ch //= 2
    if ch == b_per_w and ch >= 16:
        ch //= 2
    assert ch >= 8 and ch % 8 == 0 and b_per_w % ch == 0
    nch = b_per_w // ch
    mesh = plsc.VectorSubcoreMesh(core_axis_name="c", subcore_axis_name="s")

    @functools.partial(
        pl.kernel, mesh=mesh,
        out_type=jax.ShapeDtypeStruct((m, d), jnp.float32),
        scratch_types=[
            pltpu.VMEM((ch,), jnp.int32),
            pltpu.VMEM((ch,), jnp.int32),
            pltpu.VMEM((ch, d), jnp.float32),
            pltpu.VMEM((ch, d), jnp.float32),
            pltpu.SemaphoreType.DMA,
            pltpu.SemaphoreType.DMA,
            pltpu.SemaphoreType.DMA,
            pltpu.SemaphoreType.DMA,
        ],
    )
    def k(table_hbm, idx_hbm, out_hbm, iv0, iv1, r0, r1, g0, g1, w0, w1):
        ivs, rows = (iv0, iv1), (r0, r1)
        gs, ws = (g0, g1), (w0, w1)
        wid = lax.axis_index("s") * 2 + lax.axis_index("c")
        gh, wh = {}, {}

        def start_gather(j):
            base = wid * b_per_w + j * ch
            pltpu.sync_copy(idx_hbm.at[pl.ds(base, ch)], ivs[j % 2])
            gh[j] = pltpu.async_copy(table_hbm.at[ivs[j % 2]], rows[j % 2],
                                     gs[j % 2])

        def start_wb(j):
            base = wid * b_per_w + j * ch
            wh[j] = pltpu.async_copy(rows[j % 2], out_hbm.at[pl.ds(base, ch)],
                                     ws[j % 2])

        start_gather(0)
        for j in range(nch):
            if j + 1 < nch:
                if j >= 1:
                    wh[j - 1].wait()
                start_gather(j + 1)
            gh[j].wait()
            start_wb(j)
        wh[nch - 1].wait()
        if nch >= 2:
            wh[nch - 2].wait()

    return k


def _sc_gather(table, idx):
    m = idx.shape[0]
    pad = (-m) % 256
    if pad:
        idx = jnp.concatenate([idx, jnp.zeros((pad,), jnp.int32)])
    out = _sc_gather_fn(table.shape[0], table.shape[1], m + pad)(table, idx)
    return out[:m] if pad else out


def _pad128(p2):
    return jnp.pad(p2, ((0, 0), (0, 125)))


def _lin_kernel(x_ref, w_ref, b_ref, g_ref, bt_ref, o_ref, *, relu):
    y = _dot(x_ref[...], w_ref[...])
    y = (y + b_ref[...]) * g_ref[...] + bt_ref[...]
    if relu:
        y = jnp.maximum(y, 0.0)
    o_ref[...] = y


def _lin_res_kernel(x_ref, w_ref, b_ref, g_ref, bt_ref, r_ref, o_ref, *, relu):
    y = _dot(x_ref[...], w_ref[...])
    y = (y + b_ref[...]) * g_ref[...] + bt_ref[...] + r_ref[...]
    if relu:
        y = jnp.maximum(y, 0.0)
    o_ref[...] = y


def _lin_call(x, w, b=None, bn=None, relu=False, res=None):
    lead = x.shape[:-1]
    cin = x.shape[-1]
    cout = w.shape[-1]
    rows = int(math.prod(lead))
    x2 = x.reshape(rows, cin)
    rb = min(rows, 1024)
    b2 = jnp.zeros((1, cout), jnp.float32) if b is None else b.reshape(1, cout)
    if bn is None:
        g2 = jnp.ones((1, cout), jnp.float32)
        bt2 = jnp.zeros((1, cout), jnp.float32)
    else:
        g2 = bn['g'].reshape(1, cout)
        bt2 = bn['b'].reshape(1, cout)
    full = pl.BlockSpec((1, cout), lambda i: (0, 0))
    wspec = pl.BlockSpec((cin, cout), lambda i: (0, 0))
    rspec = pl.BlockSpec((rb, cout), lambda i: (i, 0))
    xspec = pl.BlockSpec((rb, cin), lambda i: (i, 0))
    if res is None:
        fn = functools.partial(_lin_kernel, relu=relu)
        args = (x2, w, b2, g2, bt2)
        specs = [xspec, wspec, full, full, full]
    else:
        fn = functools.partial(_lin_res_kernel, relu=relu)
        args = (x2, w, b2, g2, bt2, res.reshape(rows, cout))
        specs = [xspec, wspec, full, full, full, rspec]
    y = pl.pallas_call(
        fn,
        grid=(rows // rb,),
        in_specs=specs,
        out_specs=rspec,
        out_shape=jax.ShapeDtypeStruct((rows, cout), jnp.float32),
    )(*args)
    return y.reshape(lead + (cout,))


def _knn_kernel(q_ref, st_ref, idx_ref, dst_ref, *, k, ns):
    q = q_ref[0]
    st = st_ref[0]
    boff = pl.program_id(0) * ns
    qq = jnp.sum(q * q, axis=1, keepdims=True)
    ss = jnp.sum(st * st, axis=0, keepdims=True)
    d = qq + ss - 2.0 * _dot(q, st)
    iota = lax.broadcasted_iota(jnp.int32, d.shape, 1)
    for j in range(k):
        m = jnp.min(d, axis=1, keepdims=True)
        ij = jnp.min(jnp.where(d == m, iota, ns), axis=1, keepdims=True)
        idx_ref[0, :, j:j + 1] = ij + boff
        dst_ref[0, :, j:j + 1] = jnp.maximum(m, 0.0)
        if j + 1 < k:
            d = jnp.where(iota == ij, jnp.inf, d)


def _knn_call(q, s, k):
    bsz, nq, _ = q.shape
    ns = s.shape[1]
    st = jnp.swapaxes(s, 1, 2)
    qb = min(nq, max(64, _pow2_floor((1 << 20) // ns)))
    idx, dst = pl.pallas_call(
        functools.partial(_knn_kernel, k=k, ns=ns),
        grid=(bsz, nq // qb),
        in_specs=[
            pl.BlockSpec((1, qb, 3), lambda b, i: (b, i, 0)),
            pl.BlockSpec((1, 3, ns), lambda b, i: (b, 0, 0)),
        ],
        out_specs=[
            pl.BlockSpec((1, qb, k), lambda b, i: (b, i, 0)),
            pl.BlockSpec((1, qb, k), lambda b, i: (b, i, 0)),
        ],
        out_shape=[
            jax.ShapeDtypeStruct((bsz, nq, k), jnp.int32),
            jax.ShapeDtypeStruct((bsz, nq, k), jnp.float32),
        ],
    )(q, st)
    return idx, dst


def _fps_kernel(p_ref, o_ref, *, bsz, m, n8):
    xyz = [[p_ref[b, c] for c in range(3)] for b in range(bsz)]
    fiota = (lax.broadcasted_iota(jnp.int32, (8, n8), 0) * n8
             + lax.broadcasted_iota(jnp.int32, (8, n8), 1))
    n = 8 * n8

    def red11(v, op):
        return op(op(v, axis=1, keepdims=True), axis=0, keepdims=True)

    def coord(v, nxt):
        return red11(jnp.where(fiota == nxt, v, 0.0), jnp.sum)

    zero = jnp.zeros((1, 1), jnp.int32)
    last0 = [[coord(xyz[b][c], zero) for c in range(3)] for b in range(bsz)]
    for b in range(bsz):
        o_ref[0:1, b:b + 1] = zero + b * n

    def body(i, st):
        last, dd = st
        new_last = []
        new_dd = []
        for b in range(bsz):
            dx = xyz[b][0] - last[b][0]
            dy = xyz[b][1] - last[b][1]
            dz = xyz[b][2] - last[b][2]
            d = (dx * dx + dy * dy) + dz * dz
            db = jnp.minimum(dd[b], d)
            mx = red11(db, jnp.max)
            nxt = red11(jnp.where(db == mx, fiota, n), jnp.min)
            o_ref[pl.ds(i, 1), b:b + 1] = nxt + b * n
            new_last.append([coord(xyz[b][c], nxt) for c in range(3)])
            new_dd.append(db)
        return new_last, new_dd

    dd0 = [jnp.full((8, n8), 1e10, jnp.float32) for _ in range(bsz)]
    lax.fori_loop(1, m, body, (last0, dd0))


def _fps_call(p, m):
    bsz, n, _ = p.shape
    n8 = n // 8
    pt = jnp.swapaxes(p, 1, 2).reshape(bsz, 3, 8, n8)
    out = pl.pallas_call(
        functools.partial(_fps_kernel, bsz=bsz, m=m, n8=n8),
        in_specs=[pl.BlockSpec(memory_space=pltpu.VMEM)],
        out_specs=pl.BlockSpec(memory_space=pltpu.VMEM),
        out_shape=jax.ShapeDtypeStruct((m, bsz), jnp.int32),
    )(pt)
    return out.T


def _l1qkv_kernel(x_ref, w1_ref, g1_ref, t1_ref, wq_ref, bq_ref,
                  wkv_ref, bkv_ref, xq_ref, kv_ref):
    y = _dot(x_ref[...], w1_ref[...])
    y = jnp.maximum(y * g1_ref[...] + t1_ref[...], 0.0)
    xq_ref[...] = _dot(y, wq_ref[...]) + bq_ref[...]
    kv_ref[...] = _dot(y, wkv_ref[...]) + bkv_ref[...]


def _l1qkv_call(x, pr):
    bsz, n, c = x.shape
    rows = bsz * n
    rb = min(rows, 512)
    ptl = pr['ptl']
    wkv = jnp.concatenate([ptl['k']['W'], ptl['v']['W']], axis=1)
    bkv = jnp.concatenate([ptl['k']['b'], ptl['v']['b']]).reshape(1, 2 * c)
    full = lambda shape: pl.BlockSpec(shape, lambda i: (0, 0))
    xq, kv = pl.pallas_call(
        _l1qkv_kernel,
        grid=(rows // rb,),
        in_specs=[
            pl.BlockSpec((rb, c), lambda i: (i, 0)),
            full((c, c)), full((1, c)), full((1, c)),
            full((c, c)), full((1, c)),
            full((c, 2 * c)), full((1, 2 * c)),
        ],
        out_specs=[
            pl.BlockSpec((rb, c), lambda i: (i, 0)),
            pl.BlockSpec((rb, 2 * c), lambda i: (i, 0)),
        ],
        out_shape=[
            jax.ShapeDtypeStruct((rows, c), jnp.float32),
            jax.ShapeDtypeStruct((rows, 2 * c), jnp.float32),
        ],
    )(x.reshape(rows, c), pr['l1']['W'], pr['bn1']['g'].reshape(1, c),
      pr['bn1']['b'].reshape(1, c), ptl['q']['W'],
      ptl['q']['b'].reshape(1, c), wkv, bkv)
    return xq, kv


def _attn_kernel(xq_ref, qp_ref, g_ref, gp_ref, res_ref,
                 p1w_ref, p1b_ref, pg_ref, pb_ref, p2w_ref, p2b_ref,
                 g1_ref, t1_ref, w1w_ref, w1b_ref,
                 g2_ref, t2_ref, w2w_ref, w2b_ref,
                 bg_ref, bb_ref, l3w_ref, g3_ref, b3_ref,
                 o_ref, *, k, c, share):
    rb = o_ref.shape[0]
    rbk = rb * k
    g = g_ref[...]
    xk = g[:, 0:c]
    xv = g[:, c:2 * c]
    gp = gp_ref[...][:, 0:3]
    rel = (gp.reshape(rb, k, 3)
           - qp_ref[...].reshape(rb, 1, 3)).reshape(rbk, 3)
    t = _dot(rel, p1w_ref[...]) + p1b_ref[...]
    t = jnp.maximum(t * pg_ref[...] + pb_ref[...], 0.0)
    pe = _dot(t, p2w_ref[...]) + p2b_ref[...]
    xq = xq_ref[...]
    w3 = (xk.reshape(rb, k, c) - xq.reshape(rb, 1, c)
          + pe.reshape(rb, k, c))
    w = w3.reshape(rbk, c)
    w = jnp.maximum(w * g1_ref[...] + t1_ref[...], 0.0)
    w = _dot(w, w1w_ref[...]) + w1b_ref[...]
    w = jnp.maximum(w * g2_ref[...] + t2_ref[...], 0.0)
    w = _dot(w, w2w_ref[...]) + w2b_ref[...]
    c8 = c // share
    w3 = w.reshape(rb, k, c8)
    w3 = w3 - jnp.max(w3, axis=1, keepdims=True)
    e = jnp.exp(w3)
    sm = e / jnp.sum(e, axis=1, keepdims=True)
    v3 = (xv + pe).reshape(rb, k, c)
    wfull = jnp.concatenate([sm] * share, axis=2)
    o = jnp.sum(v3 * wfull, axis=1)
    att = jnp.maximum(o * bg_ref[...] + bb_ref[...], 0.0)
    z = _dot(att, l3w_ref[...])
    z = z * g3_ref[...] + b3_ref[...] + res_ref[...]
    o_ref[...] = jnp.maximum(z, 0.0)


def _attn_call(xq, qp, g, gp, res, pr, share, k):
    rows, c = xq.shape
    ptl = pr['ptl']
    c8 = c // share
    rb = min(rows, max(64, _pow2_floor((1 << 19) // (k * c))))
    full = lambda shape: pl.BlockSpec(shape, lambda i: (0, 0))

    def row(v):
        return v.reshape(1, -1)

    args = (xq, qp, g, gp, res,
            ptl['p1']['W'], row(ptl['p1']['b']), row(ptl['pbn']['g']),
            row(ptl['pbn']['b']), ptl['p2']['W'], row(ptl['p2']['b']),
            row(ptl['wbn1']['g']), row(ptl['wbn1']['b']),
            ptl['w1']['W'], row(ptl['w1']['b']),
            row(ptl['wbn2']['g']), row(ptl['wbn2']['b']),
            ptl['w2']['W'], row(ptl['w2']['b']),
            row(pr['bn2']['g']), row(pr['bn2']['b']),
            pr['l3']['W'], row(pr['bn3']['g']), row(pr['bn3']['b']))
    specs = [
        pl.BlockSpec((rb, c), lambda i: (i, 0)),
        pl.BlockSpec((rb, 3), lambda i: (i, 0)),
        pl.BlockSpec((rb * k, 2 * c), lambda i: (i, 0)),
        pl.BlockSpec((rb * k, 128), lambda i: (i, 0)),
        pl.BlockSpec((rb, c), lambda i: (i, 0)),
        full((3, 3)), full((1, 3)), full((1, 3)), full((1, 3)),
        full((3, c)), full((1, c)),
        full((1, c)), full((1, c)),
        full((c, c8)), full((1, c8)),
        full((1, c8)), full((1, c8)),
        full((c8, c8)), full((1, c8)),
        full((1, c)), full((1, c)),
        full((c, c)), full((1, c)), full((1, c)),
    ]
    return pl.pallas_call(
        functools.partial(_attn_kernel, k=k, c=c, share=share),
        grid=(rows // rb,),
        in_specs=specs,
        out_specs=pl.BlockSpec((rb, c), lambda i: (i, 0)),
        out_shape=jax.ShapeDtypeStruct((rows, c), jnp.float32),
    )(*args)


def _glm_kernel(g_ref, qp_ref, w_ref, gg_ref, bt_ref, o_ref, *, ns, cin):
    mb = o_ref.shape[0]
    g = g_ref[...]
    rel = (g[:, cin:cin + 3].reshape(mb, ns, 3)
           - qp_ref[...].reshape(mb, 1, 3)).reshape(mb * ns, 3)
    x = jnp.concatenate([rel, g[:, 0:cin]], axis=1)
    y = _dot(x, w_ref[...])
    y = jnp.maximum(y * gg_ref[...] + bt_ref[...], 0.0)
    o_ref[...] = jnp.max(y.reshape(mb, ns, y.shape[-1]), axis=1)


def _glm_call(g, qp, w, bn, ns, cin):
    rows = qp.shape[0]
    cout = w.shape[-1]
    gw = g.shape[-1]
    mb = min(rows, max(8, _pow2_floor((1 << 19) // (ns * max(gw, cout)))))
    full = lambda shape: pl.BlockSpec(shape, lambda i: (0, 0))
    y = pl.pallas_call(
        functools.partial(_glm_kernel, ns=ns, cin=cin),
        grid=(rows // mb,),
        in_specs=[
            pl.BlockSpec((mb * ns, gw), lambda i: (i, 0)),
            pl.BlockSpec((mb, 3), lambda i: (i, 0)),
            full((cin + 3, cout)),
            full((1, cout)), full((1, cout)),
        ],
        out_specs=pl.BlockSpec((mb, cout), lambda i: (i, 0)),
        out_shape=jax.ShapeDtypeStruct((rows, cout), jnp.float32),
    )(g, qp, w, bn['g'].reshape(1, cout), bn['b'].reshape(1, cout))
    return y


def _interp_kernel(a_ref, f_ref, d_ref, o_ref, *, c):
    r = 1.0 / (d_ref[...] + 1e-8)
    w = r / jnp.sum(r, axis=1, keepdims=True)
    f = f_ref[...][:, :, 0:c]
    o_ref[...] = a_ref[...] + jnp.sum(f * w[:, :, None], axis=1)


def _interp_call(a, f3, d):
    bsz, n, c = a.shape
    cg = f3.shape[-1]
    rows = bsz * n
    rb = min(rows, 1024)
    y = pl.pallas_call(
        functools.partial(_interp_kernel, c=c),
        grid=(rows // rb,),
        in_specs=[
            pl.BlockSpec((rb, c), lambda i: (i, 0)),
            pl.BlockSpec((rb, 3, cg), lambda i: (i, 0, 0)),
            pl.BlockSpec((rb, 3), lambda i: (i, 0)),
        ],
        out_specs=pl.BlockSpec((rb, c), lambda i: (i, 0)),
        out_shape=jax.ShapeDtypeStruct((rows, c), jnp.float32),
    )(a.reshape(rows, c), f3.reshape(rows, 3, cg), d.reshape(rows, 3))
    return y.reshape(bsz, n, c)


def _head_kernel(x_ref, w_ref, b_ref, o_ref, *, n):
    mean = jnp.sum(x_ref[0], axis=0, keepdims=True) / n
    o_ref[0] = jnp.maximum(_dot(mean, w_ref[...]) + b_ref[...], 0.0)


def _head_call(x, w, b):
    bsz, n, c = x.shape
    return pl.pallas_call(
        functools.partial(_head_kernel, n=n),
        grid=(bsz,),
        in_specs=[
            pl.BlockSpec((1, n, c), lambda b: (b, 0, 0)),
            pl.BlockSpec((c, c), lambda b: (0, 0)),
            pl.BlockSpec((1, c), lambda b: (0, 0)),
        ],
        out_specs=pl.BlockSpec((1, 1, c), lambda b: (b, 0, 0)),
        out_shape=jax.ShapeDtypeStruct((bsz, 1, c), jnp.float32),
    )(x, w, b.reshape(1, c))


def _block_apply(pr, share, ns, gp, qp, x, knn_idx):
    bsz, n, c = x.shape
    rows = bsz * n
    xq, kv = _l1qkv_call(x, pr)
    g = _sc_gather(kv, knn_idx.reshape(-1))
    out = _attn_call(xq, qp, g, gp, x.reshape(rows, c), pr, share, ns)
    return out.reshape(bsz, n, c)


def _td_down(pr, ns, pp, pp128, x):
    bsz, n, c = x.shape
    m = n // 4
    fidx = _fps_call(pp, m)
    np_rows = _sc_gather(pp128, fidx.reshape(-1))[:, :3]
    np_ = np_rows.reshape(bsz, m, 3)
    idx, _ = _knn_call(np_, pp, ns)
    gw = -(-(c + 3) // 128) * 128
    table = jnp.pad(
        jnp.concatenate([x.reshape(bsz * n, c), pp.reshape(bsz * n, 3)], axis=1),
        ((0, 0), (0, gw - c - 3)))
    g = _sc_gather(table, idx.reshape(-1))
    y = _glm_call(g, np_rows, pr['lin']['W'], pr['bn'], ns, c)
    return np_, y.reshape(bsz, m, -1)


def _tu_apply(pr, p1, x1, p2, x2):
    bsz, n1 = p1.shape[:2]
    a = _lin_call(x1, pr['l1']['W'], pr['l1']['b'], pr['bn1'], relu=True)
    w2, b2 = pr['l2']['W'], pr['l2']['b']
    bn2 = pr['bn2']
    c = w2.shape[-1]
    if c < 128:
        pad = 128 - c
        w2 = jnp.pad(w2, ((0, 0), (0, pad)))
        b2 = jnp.pad(b2, (0, pad))
        bn2 = {'g': jnp.pad(bn2['g'], (0, pad)), 'b': jnp.pad(bn2['b'], (0, pad))}
    bf = _lin_call(x2, w2, b2, bn2, relu=True)
    idx, dst = _knn_call(p1, p2, 3)
    cg = bf.shape[-1]
    f = _sc_gather(bf.reshape(-1, cg), idx.reshape(-1))
    return _interp_call(a, f.reshape(bsz, n1, 3, cg), dst)


def _tu_head(pr, x):
    bsz, n, c = x.shape
    t = _head_call(x, pr['l2']['W'], pr['l2']['b'])
    y = jnp.concatenate([x, jnp.broadcast_to(t, (bsz, n, c))], -1)
    return _lin_call(y, pr['l1']['W'], None, pr['bn1'], relu=True)


def kernel(p, x, params):
    bsz, n, _ = p.shape
    x0 = jnp.concatenate([p, x], -1)
    p1 = p
    p1f = p1.reshape(bsz * n, 3)
    p116 = _pad128(p1f)
    h1 = _lin_call(x0, params['enc1']['td']['lin']['W'], None,
                   params['enc1']['td']['bn'], relu=True)
    knn1, _ = _knn_call(p1, p1, 8)
    gp1 = _sc_gather(p116, knn1.reshape(-1))
    h1 = _block_apply(params['enc1']['b0'], 8, 8, gp1, p1f, h1, knn1)
    p2, h2 = _td_down(params['enc2']['td'], 16, p1, p116, h1)
    p2f = p2.reshape(-1, 3)
    p216 = _pad128(p2f)
    knn2, _ = _knn_call(p2, p2, 16)
    gp2 = _sc_gather(p216, knn2.reshape(-1))
    h2 = _block_apply(params['enc2']['b0'], 8, 16, gp2, p2f, h2, knn2)
    p3, h3 = _td_down(params['enc3']['td'], 16, p2, p216, h2)
    p3f = p3.reshape(-1, 3)
    p316 = _pad128(p3f)
    knn3, _ = _knn_call(p3, p3, 16)
    gp3 = _sc_gather(p316, knn3.reshape(-1))
    h3 = _block_apply(params['enc3']['b0'], 8, 16, gp3, p3f, h3, knn3)
    p4, h4 = _td_down(params['enc4']['td'], 16, p3, p316, h3)
    p4f = p4.reshape(-1, 3)
    p416 = _pad128(p4f)
    knn4, _ = _knn_call(p4, p4, 16)
    gp4 = _sc_gather(p416, knn4.reshape(-1))
    h4 = _block_apply(params['enc4']['b0'], 8, 16, gp4, p4f, h4, knn4)
    d4 = _tu_head(params['dec4']['tu'], h4)
    d4 = _block_apply(params['dec4']['b0'], 8, 16, gp4, p4f, d4, knn4)
    d3 = _tu_apply(params['dec3']['tu'], p3, h3, p4, d4)
    d3 = _block_apply(params['dec3']['b0'], 8, 16, gp3, p3f, d3, knn3)
    d2 = _tu_apply(params['dec2']['tu'], p2, h2, p3, d3)
    d2 = _block_apply(params['dec2']['b0'], 8, 16, gp2, p2f, d2, knn2)
    d1 = _tu_apply(params['dec1']['tu'], p1, h1, p2, d2)
    d1 = _block_apply(params['dec1']['b0'], 8, 8, gp1, p1f, d1, knn1)
    return (d4, d3, d2, d1)
